# baseline (speedup 1.0000x reference)
import jax
import jax.numpy as jnp
from jax.experimental import pallas as pl
from jax.experimental.pallas import tpu as pltpu

_BN_EPS = 1e-5
_N_OUT = 2
_G = 8
_W1 = 80
_PK = ((_W1, 3), (48, 3), (32, 3), (11, 1), (7, 1))

_L = (
    (3, 3, 16, 74, 72, 36),
    (3, 16, 32, 38, 36, 18),
    (3, 32, 64, 20, 18, 9),
    (2, 64, 128, 11, 10, 5),
    (2, 128, 32, 7, 6, 3),
)


def _rows(hp, ho):
    return (ho - 1) * hp + ho


def _body(x_ref, w1_ref, b1_ref, w2_ref, b2_ref, w3_ref, b3_ref,
          w4_ref, b4_ref, w5_ref, b5_ref, wfa_ref, wfb_ref, gsel_ref, fcb_ref,
          out_ref, *scr):
    bf16 = jnp.bfloat16

    def leaky(v):
        return jnp.where(v > 0, v, 0.01 * v)

    def conv(in_ref, w_ref, b_ref, act_ref, k, pitch, r, n_dj):
        acc = None
        taps = ([(di, 0) for di in range(k)] if n_dj > 1 else
                [(di, dj) for di in range(k) for dj in range(k)])
        for s, (di, dj) in enumerate(taps):
            part = jnp.dot(in_ref[pl.ds(di * pitch + dj, r), :],
                           w_ref[s],
                           preferred_element_type=jnp.float32)
            acc = part if acc is None else acc + part
        a = leaky(acc + b_ref[...])
        for t in range(act_ref.shape[0]):
            act_ref[t] = a[:, 128 * t:128 * (t + 1)]

    def pool2x2(act_ref, t, sp, ho2, i2):
        s = 2 * i2 * sp
        a00 = act_ref[t, pl.ds(s, ho2, stride=2), :]
        a01 = act_ref[t, pl.ds(s + 1, ho2, stride=2), :]
        a10 = act_ref[t, pl.ds(s + sp, ho2, stride=2), :]
        a11 = act_ref[t, pl.ds(s + sp + 1, ho2, stride=2), :]
        return jnp.maximum(jnp.maximum(a00, a01), jnp.maximum(a10, a11))

    def pool_pad(act_ref, sp, ho2, dst_ref, dst_p, dst_w, n_dj, lw):
        zrow = jnp.zeros((1, 128), bf16)
        for i2 in range(ho2):
            for t in range(act_ref.shape[0]):
                hm = pool2x2(act_ref, t, sp, ho2, i2).astype(bf16)
                row = jnp.concatenate([zrow, hm, zrow], axis=0)
                for dj in range(n_dj):
                    c0 = dj * lw + 128 * t
                    dst_ref[pl.ds((i2 + 1) * dst_p - dj, dst_w),
                            c0:c0 + 128] = row

    def zero_tb(dst_ref, hp, pitch):
        z = jnp.zeros((hp, dst_ref.shape[1]), bf16)
        dst_ref[pl.ds(0, hp), :] = z
        ext = 2 if pitch > hp else 0
        z2 = jnp.zeros((pitch + ext, dst_ref.shape[1]), bf16)
        dst_ref[pl.ds((hp - 1) * pitch - ext, pitch + ext), :] = z2

    for g2 in range(2):
        in1, act1, in2, act2, in3, act3, in4, act4, in5, act5, pool5 = (
            scr[11 * g2:11 * (g2 + 1)])

        xcm = x_ref[0, g2]
        lr = _W1 * _L[0][3] - 2
        xsh = jnp.concatenate([xcm[:, 0:lr], xcm[:, 1:lr + 1],
                               xcm[:, 2:lr + 2]], axis=0)
        in1[pl.ds(0, lr), :] = jnp.swapaxes(xsh, 0, 1).astype(bf16)

        r1 = (_L[0][4] - 1) * _W1 + _L[0][4]
        acc1 = None
        for di in range(3):
            part = jnp.dot(in1[pl.ds(di * _W1, r1), :], w1_ref[di],
                           preferred_element_type=jnp.float32)
            acc1 = part if acc1 is None else acc1 + part
        act1[0] = leaky(acc1 + b1_ref[...])

        ins = (None, in2, in3, in4, in5)
        acts = (act1, act2, act3, act4, act5)
        ws = (None, w2_ref, w3_ref, w4_ref, w5_ref)
        bs = (None, b2_ref, b3_ref, b4_ref, b5_ref)

        for i in range(5):
            k, cin, _, hp, ho, ho2 = _L[i]
            pitch, n_dj = _PK[i]
            if i > 0:
                conv(ins[i], ws[i], bs[i], acts[i], k, pitch,
                     (ho - 1) * pitch + ho, n_dj)
            if i < 4:
                nhp = _L[i + 1][3]
                npitch, nn_dj = _PK[i + 1]
                zero_tb(ins[i + 1], nhp, npitch)
                pool_pad(acts[i], pitch, ho2, ins[i + 1], npitch, nhp,
                         nn_dj, _G * _L[i + 1][1])
            else:
                for i2 in range(ho2):
                    for t in range(acts[i].shape[0]):
                        pool5[pl.ds(i2 * ho2, ho2),
                              128 * t:128 * (t + 1)] = (
                            pool2x2(acts[i], t, pitch, ho2, i2).astype(bf16))

        v = pool5[...].astype(jnp.float32)
        cs0 = jnp.sum(v * wfa_ref[...], axis=0, keepdims=True)
        cs1 = jnp.sum(v * wfb_ref[...], axis=0, keepdims=True)
        cs = jnp.concatenate([cs0, cs1], axis=0)
        logits = jnp.dot(cs, gsel_ref[...],
                         preferred_element_type=jnp.float32) + fcb_ref[...]
        out_ref[0, g2] = jax.nn.sigmoid(logits)


def kernel(c1_w, c1_b, c1_g, c1_beta, c1_m, c1_v,
           c2_w, c2_b, c2_g, c2_beta, c2_m, c2_v,
           c3_w, c3_b, c3_g, c3_beta, c3_m, c3_v,
           c4_w, c4_b, c4_g, c4_beta, c4_m, c4_v,
           c5_w, c5_b, c5_g, c5_beta, c5_m, c5_v,
           fc_w, fc_b, x):
    B = x.shape[0]
    bf16 = jnp.bfloat16

    def fold(w, b, g, beta, m, v):
        s = g * jax.lax.rsqrt(v + _BN_EPS)
        return w * s, ((b - m) * s + beta).reshape(1, -1)

    fw1, fb1 = fold(c1_w, c1_b, c1_g, c1_beta, c1_m, c1_v)
    fw2, fb2 = fold(c2_w, c2_b, c2_g, c2_beta, c2_m, c2_v)
    fw3, fb3 = fold(c3_w, c3_b, c3_g, c3_beta, c3_m, c3_v)
    fw4, fb4 = fold(c4_w, c4_b, c4_g, c4_beta, c4_m, c4_v)
    fw5, fb5 = fold(c5_w, c5_b, c5_g, c5_beta, c5_m, c5_v)

    def blockdiag(w, ksz, cin, cout):
        slab = w.reshape(ksz * ksz, cin, cout)
        out = jnp.zeros((ksz * ksz, _G * cin, _G * cout), jnp.float32)
        for g in range(_G):
            out = out.at[:, g * cin:(g + 1) * cin,
                         g * cout:(g + 1) * cout].set(slab)
        return out.astype(bf16)

    def packdiag(w, ksz, cin, cout):
        out = jnp.zeros((ksz, ksz * _G * cin, _G * cout), jnp.float32)
        for dj in range(ksz):
            for g in range(_G):
                out = out.at[:, dj * _G * cin + g * cin:
                             dj * _G * cin + (g + 1) * cin,
                             g * cout:(g + 1) * cout].set(w[:, dj])
        return out.astype(bf16)

    w1s = packdiag(fw1, 3, _L[0][1], _L[0][2])
    w2s = packdiag(fw2, 3, _L[1][1], _L[1][2])
    w3s = packdiag(fw3, 3, _L[2][1], _L[2][2])
    w4s = blockdiag(fw4, 2, _L[3][1], _L[3][2])
    w5s = blockdiag(fw5, 2, _L[4][1], _L[4][2])
    tb1 = jnp.tile(fb1, (1, _G))
    tb2 = jnp.tile(fb2, (1, _G))
    tb3 = jnp.tile(fb3, (1, _G))
    tb4 = jnp.tile(fb4, (1, _G))
    tb5 = jnp.tile(fb5, (1, _G))

    hp1 = _L[0][3]
    Bp = -(-B // (2 * _G)) * (2 * _G)
    xq = jnp.pad(x, ((0, Bp - B), (0, 0), (1, 1), (1, _W1 - 73)))
    xflat = xq.reshape(Bp // (2 * _G), 2, _G * _L[0][1], hp1 * _W1)

    fs = _L[4][5] ** 2
    c5o = _L[4][2]
    wf = fc_w.reshape(c5o, fs, _N_OUT)
    wfa = jnp.tile(wf[:, :, 0].T, (1, _G))
    wfb = jnp.tile(wf[:, :, 1].T, (1, _G))
    gsel = jnp.repeat(jnp.eye(_G, dtype=jnp.float32), c5o, axis=0)
    fcb = jnp.tile(fc_b.reshape(_N_OUT, 1), (1, _G))

    const2 = lambda shape: pl.BlockSpec(shape, lambda b: (0, 0))
    const3 = lambda shape: pl.BlockSpec(shape, lambda b: (0, 0, 0))

    out = pl.pallas_call(
        _body,
        out_shape=jax.ShapeDtypeStruct((Bp // (2 * _G), 2, _N_OUT, _G),
                                       jnp.float32),
        grid=(Bp // (2 * _G),),
        in_specs=[
            pl.BlockSpec((1, 2, _G * _L[0][1], hp1 * _W1),
                         lambda b: (b, 0, 0, 0)),
            const3(w1s.shape), const2(tb1.shape),
            const3(w2s.shape), const2(tb2.shape),
            const3(w3s.shape), const2(tb3.shape),
            const3(w4s.shape), const2(tb4.shape),
            const3(w5s.shape), const2(tb5.shape),
            const2(wfa.shape), const2(wfb.shape),
            const2(gsel.shape), const2(fcb.shape),
        ],
        out_specs=pl.BlockSpec((1, 2, _N_OUT, _G), lambda b: (b, 0, 0, 0)),
        scratch_shapes=[
            pltpu.VMEM((_L[0][3] * _W1, 3 * _G * _L[0][1]), bf16),
            pltpu.VMEM((1, (_L[0][4] - 1) * _W1 + _L[0][4], 128),
                       jnp.float32),
            pltpu.VMEM((_L[1][3] * _PK[1][0], 3 * _G * _L[1][1]), bf16),
            pltpu.VMEM((_G * _L[1][2] // 128,
                        (_L[1][4] - 1) * _PK[1][0] + _L[1][4], 128),
                       jnp.float32),
            pltpu.VMEM((_L[2][3] * _PK[2][0], 3 * _G * _L[2][1]), bf16),
            pltpu.VMEM((_G * _L[2][2] // 128,
                        (_L[2][4] - 1) * _PK[2][0] + _L[2][4], 128),
                       jnp.float32),
            pltpu.VMEM((_L[3][3] ** 2, _G * _L[3][1]), bf16),
            pltpu.VMEM((_G * _L[3][2] // 128, _rows(_L[3][3], _L[3][4]), 128),
                       jnp.float32),
            pltpu.VMEM((_L[4][3] ** 2, _G * _L[4][1]), bf16),
            pltpu.VMEM((_G * _L[4][2] // 128, _rows(_L[4][3], _L[4][4]), 128),
                       jnp.float32),
            pltpu.VMEM((fs, _G * _L[4][2]), bf16),
        ] * 2,
        compiler_params=pltpu.CompilerParams(
            dimension_semantics=("parallel",),
            vmem_limit_bytes=100 * 1024 * 1024,
        ),
    )(xflat, w1s, tb1, w2s, tb2, w3s, tb3,
      w4s, tb4, w5s, tb5, wfa, wfb, gsel, fcb)

    return jnp.transpose(out, (0, 1, 3, 2)).reshape(Bp, _N_OUT)[:B]


# speedup vs baseline: 1.1137x; 1.1137x over previous
import jax
import jax.numpy as jnp
from jax.experimental import pallas as pl
from jax.experimental.pallas import tpu as pltpu

_BN_EPS = 1e-5
_N_OUT = 2
_G = 8
_W1 = 80
_PK = ((_W1, 3), (48, 3), (32, 3), (11, 1), (7, 1))

_L = (
    (3, 3, 16, 74, 72, 36),
    (3, 16, 32, 38, 36, 18),
    (3, 32, 64, 20, 18, 9),
    (2, 64, 128, 11, 10, 5),
    (2, 128, 32, 7, 6, 3),
)


def _rows(hp, ho):
    return (ho - 1) * hp + ho


def _body(x_ref, w1_ref, b1_ref, w2_ref, b2_ref, w3_ref, b3_ref,
          w4_ref, b4_ref, w5_ref, b5_ref, wfa_ref, wfb_ref, gsel_ref, fcb_ref,
          out_ref, *scr):
    bf16 = jnp.bfloat16

    def leaky(v):
        return jnp.where(v > 0, v, 0.01 * v)

    def conv(in_ref, w_ref, b_ref, act_ref, k, pitch, r, n_dj):
        acc = None
        taps = ([(di, 0) for di in range(k)] if n_dj > 1 else
                [(di, dj) for di in range(k) for dj in range(k)])
        for s, (di, dj) in enumerate(taps):
            part = jnp.dot(in_ref[pl.ds(di * pitch + dj, r), :],
                           w_ref[s],
                           preferred_element_type=jnp.float32)
            acc = part if acc is None else acc + part
        a = leaky(acc + b_ref[...])
        for t in range(act_ref.shape[0]):
            act_ref[t] = a[:, 128 * t:128 * (t + 1)]

    def pool2x2(act_ref, t, sp, ho2, i2):
        s = 2 * i2 * sp
        a00 = act_ref[t, pl.ds(s, ho2, stride=2), :]
        a01 = act_ref[t, pl.ds(s + 1, ho2, stride=2), :]
        a10 = act_ref[t, pl.ds(s + sp, ho2, stride=2), :]
        a11 = act_ref[t, pl.ds(s + sp + 1, ho2, stride=2), :]
        return jnp.maximum(jnp.maximum(a00, a01), jnp.maximum(a10, a11))

    def pool_pad(act_ref, sp, ho2, dst_ref, dst_p, dst_w, n_dj, lw):
        zrow = jnp.zeros((1, 128), bf16)
        for i2 in range(ho2):
            for t in range(act_ref.shape[0]):
                hm = pool2x2(act_ref, t, sp, ho2, i2).astype(bf16)
                row = jnp.concatenate([zrow, hm, zrow], axis=0)
                for dj in range(n_dj):
                    c0 = dj * lw + 128 * t
                    dst_ref[pl.ds((i2 + 1) * dst_p - dj, dst_w),
                            c0:c0 + 128] = row

    def zero_tb(dst_ref, hp, pitch):
        z = jnp.zeros((hp, dst_ref.shape[1]), bf16)
        dst_ref[pl.ds(0, hp), :] = z
        ext = 2 if pitch > hp else 0
        z2 = jnp.zeros((pitch + ext, dst_ref.shape[1]), bf16)
        dst_ref[pl.ds((hp - 1) * pitch - ext, pitch + ext), :] = z2

    if True:
        g2 = 0
        in1, act1, in2, act2, in3, act3, in4, act4, in5, act5, pool5 = scr

        xcm = x_ref[0]
        lr = _W1 * _L[0][3] - 2
        xsh = jnp.concatenate([xcm[:, 0:lr], xcm[:, 1:lr + 1],
                               xcm[:, 2:lr + 2]], axis=0)
        in1[pl.ds(0, lr), :] = jnp.swapaxes(xsh, 0, 1).astype(bf16)

        r1 = (_L[0][4] - 1) * _W1 + _L[0][4]
        acc1 = None
        for di in range(3):
            part = jnp.dot(in1[pl.ds(di * _W1, r1), :], w1_ref[di],
                           preferred_element_type=jnp.float32)
            acc1 = part if acc1 is None else acc1 + part
        act1[0] = leaky(acc1 + b1_ref[...])

        ins = (None, in2, in3, in4, in5)
        acts = (act1, act2, act3, act4, act5)
        ws = (None, w2_ref, w3_ref, w4_ref, w5_ref)
        bs = (None, b2_ref, b3_ref, b4_ref, b5_ref)

        for i in range(5):
            k, cin, _, hp, ho, ho2 = _L[i]
            pitch, n_dj = _PK[i]
            if i > 0:
                conv(ins[i], ws[i], bs[i], acts[i], k, pitch,
                     (ho - 1) * pitch + ho, n_dj)
            if i < 4:
                nhp = _L[i + 1][3]
                npitch, nn_dj = _PK[i + 1]
                zero_tb(ins[i + 1], nhp, npitch)
                pool_pad(acts[i], pitch, ho2, ins[i + 1], npitch, nhp,
                         nn_dj, _G * _L[i + 1][1])
            else:
                for i2 in range(ho2):
                    for t in range(acts[i].shape[0]):
                        pool5[pl.ds(i2 * ho2, ho2),
                              128 * t:128 * (t + 1)] = (
                            pool2x2(acts[i], t, pitch, ho2, i2).astype(bf16))

        v = pool5[...].astype(jnp.float32)
        cs0 = jnp.sum(v * wfa_ref[...], axis=0, keepdims=True)
        cs1 = jnp.sum(v * wfb_ref[...], axis=0, keepdims=True)
        cs = jnp.concatenate([cs0, cs1], axis=0)
        logits = jnp.dot(cs, gsel_ref[...],
                         preferred_element_type=jnp.float32) + fcb_ref[...]
        out_ref[0] = jax.nn.sigmoid(logits)


def kernel(c1_w, c1_b, c1_g, c1_beta, c1_m, c1_v,
           c2_w, c2_b, c2_g, c2_beta, c2_m, c2_v,
           c3_w, c3_b, c3_g, c3_beta, c3_m, c3_v,
           c4_w, c4_b, c4_g, c4_beta, c4_m, c4_v,
           c5_w, c5_b, c5_g, c5_beta, c5_m, c5_v,
           fc_w, fc_b, x):
    B = x.shape[0]
    bf16 = jnp.bfloat16

    def fold(w, b, g, beta, m, v):
        s = g * jax.lax.rsqrt(v + _BN_EPS)
        return w * s, ((b - m) * s + beta).reshape(1, -1)

    fw1, fb1 = fold(c1_w, c1_b, c1_g, c1_beta, c1_m, c1_v)
    fw2, fb2 = fold(c2_w, c2_b, c2_g, c2_beta, c2_m, c2_v)
    fw3, fb3 = fold(c3_w, c3_b, c3_g, c3_beta, c3_m, c3_v)
    fw4, fb4 = fold(c4_w, c4_b, c4_g, c4_beta, c4_m, c4_v)
    fw5, fb5 = fold(c5_w, c5_b, c5_g, c5_beta, c5_m, c5_v)

    eyeg = jnp.eye(_G, dtype=jnp.float32)

    def blockdiag(w, ksz, cin, cout):
        slab = w.reshape(ksz * ksz, cin, cout)
        out = jnp.einsum('tco,gh->tgcho', slab, eyeg)
        return out.reshape(ksz * ksz, _G * cin, _G * cout).astype(bf16)

    def packdiag(w, ksz, cin, cout):
        out = jnp.einsum('djco,gh->djgcho', w, eyeg)
        return out.reshape(ksz, ksz * _G * cin, _G * cout).astype(bf16)

    w1s = packdiag(fw1, 3, _L[0][1], _L[0][2])
    w2s = packdiag(fw2, 3, _L[1][1], _L[1][2])
    w3s = packdiag(fw3, 3, _L[2][1], _L[2][2])
    w4s = blockdiag(fw4, 2, _L[3][1], _L[3][2])
    w5s = blockdiag(fw5, 2, _L[4][1], _L[4][2])
    tb1 = jnp.tile(fb1, (1, _G))
    tb2 = jnp.tile(fb2, (1, _G))
    tb3 = jnp.tile(fb3, (1, _G))
    tb4 = jnp.tile(fb4, (1, _G))
    tb5 = jnp.tile(fb5, (1, _G))

    hp1 = _L[0][3]
    Bp = -(-B // _G) * _G
    xq = jnp.pad(x, ((0, Bp - B), (0, 0), (1, 1), (1, _W1 - 73)))
    xflat = xq.reshape(Bp // _G, _G * _L[0][1], hp1 * _W1)

    fs = _L[4][5] ** 2
    c5o = _L[4][2]
    wf = fc_w.reshape(c5o, fs, _N_OUT)
    wfa = jnp.tile(wf[:, :, 0].T, (1, _G))
    wfb = jnp.tile(wf[:, :, 1].T, (1, _G))
    gsel = jnp.repeat(jnp.eye(_G, dtype=jnp.float32), c5o, axis=0)
    fcb = jnp.tile(fc_b.reshape(_N_OUT, 1), (1, _G))

    const2 = lambda shape: pl.BlockSpec(shape, lambda b: (0, 0))
    const3 = lambda shape: pl.BlockSpec(shape, lambda b: (0, 0, 0))

    out = pl.pallas_call(
        _body,
        out_shape=jax.ShapeDtypeStruct((Bp // _G, _N_OUT, _G), jnp.float32),
        grid=(Bp // _G,),
        in_specs=[
            pl.BlockSpec((1, _G * _L[0][1], hp1 * _W1), lambda b: (b, 0, 0)),
            const3(w1s.shape), const2(tb1.shape),
            const3(w2s.shape), const2(tb2.shape),
            const3(w3s.shape), const2(tb3.shape),
            const3(w4s.shape), const2(tb4.shape),
            const3(w5s.shape), const2(tb5.shape),
            const2(wfa.shape), const2(wfb.shape),
            const2(gsel.shape), const2(fcb.shape),
        ],
        out_specs=pl.BlockSpec((1, _N_OUT, _G), lambda b: (b, 0, 0)),
        scratch_shapes=[
            pltpu.VMEM((_L[0][3] * _W1, 3 * _G * _L[0][1]), bf16),
            pltpu.VMEM((1, (_L[0][4] - 1) * _W1 + _L[0][4], 128),
                       jnp.float32),
            pltpu.VMEM((_L[1][3] * _PK[1][0], 3 * _G * _L[1][1]), bf16),
            pltpu.VMEM((_G * _L[1][2] // 128,
                        (_L[1][4] - 1) * _PK[1][0] + _L[1][4], 128),
                       jnp.float32),
            pltpu.VMEM((_L[2][3] * _PK[2][0], 3 * _G * _L[2][1]), bf16),
            pltpu.VMEM((_G * _L[2][2] // 128,
                        (_L[2][4] - 1) * _PK[2][0] + _L[2][4], 128),
                       jnp.float32),
            pltpu.VMEM((_L[3][3] ** 2, _G * _L[3][1]), bf16),
            pltpu.VMEM((_G * _L[3][2] // 128, _rows(_L[3][3], _L[3][4]), 128),
                       jnp.float32),
            pltpu.VMEM((_L[4][3] ** 2, _G * _L[4][1]), bf16),
            pltpu.VMEM((_G * _L[4][2] // 128, _rows(_L[4][3], _L[4][4]), 128),
                       jnp.float32),
            pltpu.VMEM((fs, _G * _L[4][2]), bf16),
        ],
        compiler_params=pltpu.CompilerParams(
            dimension_semantics=("parallel",),
            vmem_limit_bytes=100 * 1024 * 1024,
        ),
    )(xflat, w1s, tb1, w2s, tb2, w3s, tb3,
      w4s, tb4, w5s, tb5, wfa, wfb, gsel, fcb)

    return jnp.transpose(out, (0, 2, 1)).reshape(Bp, _N_OUT)[:B]


# speedup vs baseline: 1.1539x; 1.0361x over previous
import jax
import jax.numpy as jnp
from jax.experimental import pallas as pl
from jax.experimental.pallas import tpu as pltpu

_BN_EPS = 1e-5
_N_OUT = 2
_G = 8
_W1 = 80
_PK = ((_W1, 3), (48, 3), (32, 3), (11, 1), (7, 1))

_L = (
    (3, 3, 16, 74, 72, 36),
    (3, 16, 32, 38, 36, 18),
    (3, 32, 64, 20, 18, 9),
    (2, 64, 128, 11, 10, 5),
    (2, 128, 32, 7, 6, 3),
)


def _rows(hp, ho):
    return (ho - 1) * hp + ho


def _body(x_ref, w1_ref, b1_ref, w2_ref, b2_ref, w3_ref, b3_ref,
          w4_ref, b4_ref, w5_ref, b5_ref, wfa_ref, wfb_ref, gsel_ref, fcb_ref,
          out_ref, *scr):
    bf16 = jnp.bfloat16

    def leaky(v):
        return jnp.where(v > 0, v, 0.01 * v)

    def conv(in_ref, w_ref, b_ref, act_ref, k, pitch, r, n_dj):
        acc = None
        taps = ([(di, 0) for di in range(k)] if n_dj > 1 else
                [(di, dj) for di in range(k) for dj in range(k)])
        for s, (di, dj) in enumerate(taps):
            part = jnp.dot(in_ref[pl.ds(di * pitch + dj, r), :],
                           w_ref[s],
                           preferred_element_type=jnp.float32)
            acc = part if acc is None else acc + part
        a = leaky(acc + b_ref[...])
        for t in range(act_ref.shape[0]):
            act_ref[t] = a[:, 128 * t:128 * (t + 1)]

    def pool2x2(act_ref, t, sp, ho2, i2):
        s = 2 * i2 * sp
        a00 = act_ref[t, pl.ds(s, ho2, stride=2), :]
        a01 = act_ref[t, pl.ds(s + 1, ho2, stride=2), :]
        a10 = act_ref[t, pl.ds(s + sp, ho2, stride=2), :]
        a11 = act_ref[t, pl.ds(s + sp + 1, ho2, stride=2), :]
        return jnp.maximum(jnp.maximum(a00, a01), jnp.maximum(a10, a11))

    def pool_pad(act_ref, sp, ho2, dst_ref, dst_p, dst_w, n_dj, lw):
        zrow = jnp.zeros((1, 128), bf16)
        for i2 in range(ho2):
            for t in range(act_ref.shape[0]):
                hm = pool2x2(act_ref, t, sp, ho2, i2).astype(bf16)
                row = jnp.concatenate([zrow, hm, zrow], axis=0)
                for dj in range(n_dj):
                    c0 = dj * lw + 128 * t
                    dst_ref[pl.ds((i2 + 1) * dst_p - dj, dst_w),
                            c0:c0 + 128] = row

    def zero_tb(dst_ref, hp, pitch):
        z = jnp.zeros((hp, dst_ref.shape[1]), bf16)
        dst_ref[pl.ds(0, hp), :] = z
        ext = 2 if pitch > hp else 0
        z2 = jnp.zeros((pitch + ext, dst_ref.shape[1]), bf16)
        dst_ref[pl.ds((hp - 1) * pitch - ext, pitch + ext), :] = z2

    if True:
        g2 = 0
        in1, act1, in2, act2, in3, act3, in4, act4, in5, act5, pool5 = scr

        xcm = x_ref[0]
        lr = _W1 * _L[0][3] - 2
        xsh = jnp.concatenate([xcm[:, 0:lr], xcm[:, 1:lr + 1],
                               xcm[:, 2:lr + 2]], axis=0)
        in1[pl.ds(0, lr), :] = jnp.swapaxes(xsh, 0, 1)

        r1 = (_L[0][4] - 1) * _W1 + _L[0][4]
        acc1 = None
        for di in range(3):
            part = jnp.dot(in1[pl.ds(di * _W1, r1), :], w1_ref[di],
                           preferred_element_type=jnp.float32)
            acc1 = part if acc1 is None else acc1 + part
        act1[0] = leaky(acc1 + b1_ref[...])

        ins = (None, in2, in3, in4, in5)
        acts = (act1, act2, act3, act4, act5)
        ws = (None, w2_ref, w3_ref, w4_ref, w5_ref)
        bs = (None, b2_ref, b3_ref, b4_ref, b5_ref)

        for i in range(5):
            k, cin, _, hp, ho, ho2 = _L[i]
            pitch, n_dj = _PK[i]
            if i > 0:
                conv(ins[i], ws[i], bs[i], acts[i], k, pitch,
                     (ho - 1) * pitch + ho, n_dj)
            if i < 4:
                nhp = _L[i + 1][3]
                npitch, nn_dj = _PK[i + 1]
                zero_tb(ins[i + 1], nhp, npitch)
                pool_pad(acts[i], pitch, ho2, ins[i + 1], npitch, nhp,
                         nn_dj, _G * _L[i + 1][1])
            else:
                for i2 in range(ho2):
                    for t in range(acts[i].shape[0]):
                        pool5[pl.ds(i2 * ho2, ho2),
                              128 * t:128 * (t + 1)] = (
                            pool2x2(acts[i], t, pitch, ho2, i2).astype(bf16))

        v = pool5[...].astype(jnp.float32)
        cs0 = jnp.sum(v * wfa_ref[...], axis=0, keepdims=True)
        cs1 = jnp.sum(v * wfb_ref[...], axis=0, keepdims=True)
        cs = jnp.concatenate([cs0, cs1], axis=0)
        logits = jnp.dot(cs, gsel_ref[...],
                         preferred_element_type=jnp.float32) + fcb_ref[...]
        out_ref[0] = jax.nn.sigmoid(logits)


def kernel(c1_w, c1_b, c1_g, c1_beta, c1_m, c1_v,
           c2_w, c2_b, c2_g, c2_beta, c2_m, c2_v,
           c3_w, c3_b, c3_g, c3_beta, c3_m, c3_v,
           c4_w, c4_b, c4_g, c4_beta, c4_m, c4_v,
           c5_w, c5_b, c5_g, c5_beta, c5_m, c5_v,
           fc_w, fc_b, x):
    B = x.shape[0]
    bf16 = jnp.bfloat16

    def fold(w, b, g, beta, m, v):
        s = g * jax.lax.rsqrt(v + _BN_EPS)
        return w * s, ((b - m) * s + beta).reshape(1, -1)

    fw1, fb1 = fold(c1_w, c1_b, c1_g, c1_beta, c1_m, c1_v)
    fw2, fb2 = fold(c2_w, c2_b, c2_g, c2_beta, c2_m, c2_v)
    fw3, fb3 = fold(c3_w, c3_b, c3_g, c3_beta, c3_m, c3_v)
    fw4, fb4 = fold(c4_w, c4_b, c4_g, c4_beta, c4_m, c4_v)
    fw5, fb5 = fold(c5_w, c5_b, c5_g, c5_beta, c5_m, c5_v)

    eyeg = jnp.eye(_G, dtype=jnp.float32)

    def blockdiag(w, ksz, cin, cout):
        slab = w.reshape(ksz * ksz, cin, cout)
        out = jnp.einsum('tco,gh->tgcho', slab, eyeg)
        return out.reshape(ksz * ksz, _G * cin, _G * cout).astype(bf16)

    def packdiag(w, ksz, cin, cout):
        out = jnp.einsum('djco,gh->djgcho', w, eyeg)
        return out.reshape(ksz, ksz * _G * cin, _G * cout).astype(bf16)

    w1s = packdiag(fw1, 3, _L[0][1], _L[0][2])
    w2s = packdiag(fw2, 3, _L[1][1], _L[1][2])
    w3s = packdiag(fw3, 3, _L[2][1], _L[2][2])
    w4s = blockdiag(fw4, 2, _L[3][1], _L[3][2])
    w5s = blockdiag(fw5, 2, _L[4][1], _L[4][2])
    tb1 = jnp.tile(fb1, (1, _G))
    tb2 = jnp.tile(fb2, (1, _G))
    tb3 = jnp.tile(fb3, (1, _G))
    tb4 = jnp.tile(fb4, (1, _G))
    tb5 = jnp.tile(fb5, (1, _G))

    hp1 = _L[0][3]
    Bp = -(-B // _G) * _G
    xq = jnp.pad(x.astype(bf16), ((0, Bp - B), (0, 0), (1, 1), (1, _W1 - 73)))
    xflat = xq.reshape(Bp // _G, _G * _L[0][1], hp1 * _W1)

    fs = _L[4][5] ** 2
    c5o = _L[4][2]
    wf = fc_w.reshape(c5o, fs, _N_OUT)
    wfa = jnp.tile(wf[:, :, 0].T, (1, _G))
    wfb = jnp.tile(wf[:, :, 1].T, (1, _G))
    gsel = jnp.repeat(jnp.eye(_G, dtype=jnp.float32), c5o, axis=0)
    fcb = jnp.tile(fc_b.reshape(_N_OUT, 1), (1, _G))

    const2 = lambda shape: pl.BlockSpec(shape, lambda b: (0, 0))
    const3 = lambda shape: pl.BlockSpec(shape, lambda b: (0, 0, 0))

    out = pl.pallas_call(
        _body,
        out_shape=jax.ShapeDtypeStruct((Bp // _G, _N_OUT, _G), jnp.float32),
        grid=(Bp // _G,),
        in_specs=[
            pl.BlockSpec((1, _G * _L[0][1], hp1 * _W1), lambda b: (b, 0, 0)),
            const3(w1s.shape), const2(tb1.shape),
            const3(w2s.shape), const2(tb2.shape),
            const3(w3s.shape), const2(tb3.shape),
            const3(w4s.shape), const2(tb4.shape),
            const3(w5s.shape), const2(tb5.shape),
            const2(wfa.shape), const2(wfb.shape),
            const2(gsel.shape), const2(fcb.shape),
        ],
        out_specs=pl.BlockSpec((1, _N_OUT, _G), lambda b: (b, 0, 0)),
        scratch_shapes=[
            pltpu.VMEM((_L[0][3] * _W1, 3 * _G * _L[0][1]), bf16),
            pltpu.VMEM((1, (_L[0][4] - 1) * _W1 + _L[0][4], 128),
                       jnp.float32),
            pltpu.VMEM((_L[1][3] * _PK[1][0], 3 * _G * _L[1][1]), bf16),
            pltpu.VMEM((_G * _L[1][2] // 128,
                        (_L[1][4] - 1) * _PK[1][0] + _L[1][4], 128),
                       jnp.float32),
            pltpu.VMEM((_L[2][3] * _PK[2][0], 3 * _G * _L[2][1]), bf16),
            pltpu.VMEM((_G * _L[2][2] // 128,
                        (_L[2][4] - 1) * _PK[2][0] + _L[2][4], 128),
                       jnp.float32),
            pltpu.VMEM((_L[3][3] ** 2, _G * _L[3][1]), bf16),
            pltpu.VMEM((_G * _L[3][2] // 128, _rows(_L[3][3], _L[3][4]), 128),
                       jnp.float32),
            pltpu.VMEM((_L[4][3] ** 2, _G * _L[4][1]), bf16),
            pltpu.VMEM((_G * _L[4][2] // 128, _rows(_L[4][3], _L[4][4]), 128),
                       jnp.float32),
            pltpu.VMEM((fs, _G * _L[4][2]), bf16),
        ],
        compiler_params=pltpu.CompilerParams(
            dimension_semantics=("parallel",),
            vmem_limit_bytes=100 * 1024 * 1024,
        ),
    )(xflat, w1s, tb1, w2s, tb2, w3s, tb3,
      w4s, tb4, w5s, tb5, wfa, wfb, gsel, fcb)

    return jnp.transpose(out, (0, 2, 1)).reshape(Bp, _N_OUT)[:B]


# speedup vs baseline: 1.1546x; 1.0006x over previous
import jax
import jax.numpy as jnp
from jax.experimental import pallas as pl
from jax.experimental.pallas import tpu as pltpu

_BN_EPS = 1e-5
_N_OUT = 2
_G = 8
_W1 = 80
_PK = ((_W1, 3), (48, 3), (32, 3), (11, 1), (7, 1))

_L = (
    (3, 3, 16, 74, 72, 36),
    (3, 16, 32, 38, 36, 18),
    (3, 32, 64, 20, 18, 9),
    (2, 64, 128, 11, 10, 5),
    (2, 128, 32, 7, 6, 3),
)


def _rows(hp, ho):
    return (ho - 1) * hp + ho


def _body(x_ref, w1_ref, b1_ref, w2_ref, b2_ref, w3_ref, b3_ref,
          w4_ref, b4_ref, w5_ref, b5_ref, wfa_ref, wfb_ref, gsel_ref, fcb_ref,
          out_ref, *scr):
    bf16 = jnp.bfloat16

    def leaky(v):
        return jnp.where(v > 0, v, 0.01 * v)

    def conv(in_ref, w_ref, b_ref, act_ref, k, pitch, r, n_dj):
        acc = None
        taps = ([(di, 0) for di in range(k)] if n_dj > 1 else
                [(di, dj) for di in range(k) for dj in range(k)])
        for s, (di, dj) in enumerate(taps):
            part = jnp.dot(in_ref[pl.ds(di * pitch + dj, r), :],
                           w_ref[s],
                           preferred_element_type=jnp.float32)
            acc = part if acc is None else acc + part
        a = leaky(acc + b_ref[...])
        for t in range(act_ref.shape[0]):
            act_ref[t] = a[:, 128 * t:128 * (t + 1)]

    def pool2x2(act_ref, t, sp, ho2, i2):
        s = 2 * i2 * sp
        a00 = act_ref[t, pl.ds(s, ho2, stride=2), :]
        a01 = act_ref[t, pl.ds(s + 1, ho2, stride=2), :]
        a10 = act_ref[t, pl.ds(s + sp, ho2, stride=2), :]
        a11 = act_ref[t, pl.ds(s + sp + 1, ho2, stride=2), :]
        return jnp.maximum(jnp.maximum(a00, a01), jnp.maximum(a10, a11))

    def pool_pad(act_ref, sp, ho2, dst_ref, dst_p, dst_w, n_dj, lw):
        zrow = jnp.zeros((1, 128), bf16)
        for i2 in range(ho2):
            for t in range(act_ref.shape[0]):
                hm = pool2x2(act_ref, t, sp, ho2, i2).astype(bf16)
                row = jnp.concatenate([zrow, hm, zrow], axis=0)
                for dj in range(n_dj):
                    c0 = dj * lw + 128 * t
                    dst_ref[pl.ds((i2 + 1) * dst_p - dj, dst_w),
                            c0:c0 + 128] = row

    def zero_tb(dst_ref, hp, pitch):
        z = jnp.zeros((hp, dst_ref.shape[1]), bf16)
        dst_ref[pl.ds(0, hp), :] = z
        ext = 2 if pitch > hp else 0
        z2 = jnp.zeros((pitch + ext, dst_ref.shape[1]), bf16)
        dst_ref[pl.ds((hp - 1) * pitch - ext, pitch + ext), :] = z2

    in1, act1, in2, act2, in3, act3, in4, act4, in5, act5, pool5 = scr

    xcm = x_ref[0]
    lr = _W1 * _L[0][3] - 2
    xsh = jnp.concatenate([xcm[:, 0:lr], xcm[:, 1:lr + 1],
                           xcm[:, 2:lr + 2]], axis=0)
    in1[pl.ds(0, lr), :] = jnp.swapaxes(xsh, 0, 1)

    r1 = (_L[0][4] - 1) * _W1 + _L[0][4]
    acc1 = None
    for di in range(3):
        part = jnp.dot(in1[pl.ds(di * _W1, r1), :], w1_ref[di],
                       preferred_element_type=jnp.float32)
        acc1 = part if acc1 is None else acc1 + part
    act1[0] = leaky(acc1 + b1_ref[...])

    ins = (None, in2, in3, in4, in5)
    acts = (act1, act2, act3, act4, act5)
    ws = (None, w2_ref, w3_ref, w4_ref, w5_ref)
    bs = (None, b2_ref, b3_ref, b4_ref, b5_ref)

    for i in range(5):
        k, cin, _, hp, ho, ho2 = _L[i]
        pitch, n_dj = _PK[i]
        if i > 0:
            conv(ins[i], ws[i], bs[i], acts[i], k, pitch,
                 (ho - 1) * pitch + ho, n_dj)
        if i < 4:
            nhp = _L[i + 1][3]
            npitch, nn_dj = _PK[i + 1]
            zero_tb(ins[i + 1], nhp, npitch)
            pool_pad(acts[i], pitch, ho2, ins[i + 1], npitch, nhp,
                     nn_dj, _G * _L[i + 1][1])
        else:
            for i2 in range(ho2):
                for t in range(acts[i].shape[0]):
                    pool5[pl.ds(i2 * ho2, ho2),
                          128 * t:128 * (t + 1)] = (
                        pool2x2(acts[i], t, pitch, ho2, i2).astype(bf16))

    v = pool5[...].astype(jnp.float32)
    cs0 = jnp.sum(v * wfa_ref[...], axis=0, keepdims=True)
    cs1 = jnp.sum(v * wfb_ref[...], axis=0, keepdims=True)
    cs = jnp.concatenate([cs0, cs1], axis=0)
    logits = jnp.dot(cs, gsel_ref[...],
                     preferred_element_type=jnp.float32) + fcb_ref[...]
    out_ref[0] = jax.nn.sigmoid(logits)


def kernel(c1_w, c1_b, c1_g, c1_beta, c1_m, c1_v,
           c2_w, c2_b, c2_g, c2_beta, c2_m, c2_v,
           c3_w, c3_b, c3_g, c3_beta, c3_m, c3_v,
           c4_w, c4_b, c4_g, c4_beta, c4_m, c4_v,
           c5_w, c5_b, c5_g, c5_beta, c5_m, c5_v,
           fc_w, fc_b, x):
    B = x.shape[0]
    bf16 = jnp.bfloat16

    def fold(w, b, g, beta, m, v):
        s = g * jax.lax.rsqrt(v + _BN_EPS)
        return w * s, ((b - m) * s + beta).reshape(1, -1)

    fw1, fb1 = fold(c1_w, c1_b, c1_g, c1_beta, c1_m, c1_v)
    fw2, fb2 = fold(c2_w, c2_b, c2_g, c2_beta, c2_m, c2_v)
    fw3, fb3 = fold(c3_w, c3_b, c3_g, c3_beta, c3_m, c3_v)
    fw4, fb4 = fold(c4_w, c4_b, c4_g, c4_beta, c4_m, c4_v)
    fw5, fb5 = fold(c5_w, c5_b, c5_g, c5_beta, c5_m, c5_v)

    eyeg = jnp.eye(_G, dtype=jnp.float32)

    def blockdiag(w, ksz, cin, cout):
        slab = w.reshape(ksz * ksz, cin, cout)
        out = jnp.einsum('tco,gh->tgcho', slab, eyeg)
        return out.reshape(ksz * ksz, _G * cin, _G * cout).astype(bf16)

    def packdiag(w, ksz, cin, cout):
        out = jnp.einsum('djco,gh->djgcho', w, eyeg)
        return out.reshape(ksz, ksz * _G * cin, _G * cout).astype(bf16)

    w1s = packdiag(fw1, 3, _L[0][1], _L[0][2])
    w2s = packdiag(fw2, 3, _L[1][1], _L[1][2])
    w3s = packdiag(fw3, 3, _L[2][1], _L[2][2])
    w4s = blockdiag(fw4, 2, _L[3][1], _L[3][2])
    w5s = blockdiag(fw5, 2, _L[4][1], _L[4][2])
    tb1 = jnp.tile(fb1, (1, _G))
    tb2 = jnp.tile(fb2, (1, _G))
    tb3 = jnp.tile(fb3, (1, _G))
    tb4 = jnp.tile(fb4, (1, _G))
    tb5 = jnp.tile(fb5, (1, _G))

    hp1 = _L[0][3]
    Bp = -(-B // _G) * _G
    xq = jnp.pad(x.astype(bf16), ((0, Bp - B), (0, 0), (1, 1), (1, _W1 - 73)))
    xflat = xq.reshape(Bp // _G, _G * _L[0][1], hp1 * _W1)

    fs = _L[4][5] ** 2
    c5o = _L[4][2]
    wf = fc_w.reshape(c5o, fs, _N_OUT)
    wfa = jnp.tile(wf[:, :, 0].T, (1, _G))
    wfb = jnp.tile(wf[:, :, 1].T, (1, _G))
    gsel = jnp.repeat(jnp.eye(_G, dtype=jnp.float32), c5o, axis=0)
    fcb = jnp.tile(fc_b.reshape(_N_OUT, 1), (1, _G))

    const2 = lambda shape: pl.BlockSpec(shape, lambda b: (0, 0))
    const3 = lambda shape: pl.BlockSpec(shape, lambda b: (0, 0, 0))

    out = pl.pallas_call(
        _body,
        out_shape=jax.ShapeDtypeStruct((Bp // _G, _N_OUT, _G), jnp.float32),
        grid=(Bp // _G,),
        in_specs=[
            pl.BlockSpec((1, _G * _L[0][1], hp1 * _W1), lambda b: (b, 0, 0)),
            const3(w1s.shape), const2(tb1.shape),
            const3(w2s.shape), const2(tb2.shape),
            const3(w3s.shape), const2(tb3.shape),
            const3(w4s.shape), const2(tb4.shape),
            const3(w5s.shape), const2(tb5.shape),
            const2(wfa.shape), const2(wfb.shape),
            const2(gsel.shape), const2(fcb.shape),
        ],
        out_specs=pl.BlockSpec((1, _N_OUT, _G), lambda b: (b, 0, 0)),
        scratch_shapes=[
            pltpu.VMEM((_L[0][3] * _W1, 3 * _G * _L[0][1]), bf16),
            pltpu.VMEM((1, (_L[0][4] - 1) * _W1 + _L[0][4], 128),
                       jnp.float32),
            pltpu.VMEM((_L[1][3] * _PK[1][0], 3 * _G * _L[1][1]), bf16),
            pltpu.VMEM((_G * _L[1][2] // 128,
                        (_L[1][4] - 1) * _PK[1][0] + _L[1][4], 128),
                       jnp.float32),
            pltpu.VMEM((_L[2][3] * _PK[2][0], 3 * _G * _L[2][1]), bf16),
            pltpu.VMEM((_G * _L[2][2] // 128,
                        (_L[2][4] - 1) * _PK[2][0] + _L[2][4], 128),
                       jnp.float32),
            pltpu.VMEM((_L[3][3] ** 2, _G * _L[3][1]), bf16),
            pltpu.VMEM((_G * _L[3][2] // 128, _rows(_L[3][3], _L[3][4]), 128),
                       jnp.float32),
            pltpu.VMEM((_L[4][3] ** 2, _G * _L[4][1]), bf16),
            pltpu.VMEM((_G * _L[4][2] // 128, _rows(_L[4][3], _L[4][4]), 128),
                       jnp.float32),
            pltpu.VMEM((fs, _G * _L[4][2]), bf16),
        ],
        compiler_params=pltpu.CompilerParams(
            dimension_semantics=("parallel",),
            vmem_limit_bytes=100 * 1024 * 1024,
        ),
    )(xflat, w1s, tb1, w2s, tb2, w3s, tb3,
      w4s, tb4, w5s, tb5, wfa, wfb, gsel, fcb)

    return jnp.transpose(out, (0, 2, 1)).reshape(Bp, _N_OUT)[:B]
